# Initial kernel scaffold; baseline (speedup 1.0000x reference)
import functools

import jax
import jax.numpy as jnp
from jax import lax
from jax.experimental import pallas as pl
from jax.experimental.pallas import tpu as pltpu
from jax.experimental.pallas import tpu_sc as plsc

NC, NS, LANES = 2, 16, 16
NW = NC * NS
C = 128
D = 128
NF = 8
BGRAPH = 64
ZR = 65

_f32 = jnp.float32
_i32 = jnp.int32


def _pad_edges(src, dst):
    e = src.shape[0]
    epad = -e % (NW * C)
    src = jnp.concatenate([src.astype(_i32), jnp.zeros((epad,), _i32)])
    dst = jnp.concatenate([dst.astype(_i32), jnp.full((epad,), 1 << 29, _i32)])
    return src, dst


def _segsum_call(table, src, dst, n_out, half, seg):
    n_pass = (n_out + half - 1) // half
    epw = src.shape[0] // NW
    nch = epw // C
    trash = seg - 1
    zchunk = seg // NS
    assert seg % NS == 0 and zchunk % ZR == 0

    mesh = plsc.VectorSubcoreMesh(core_axis_name="c", subcore_axis_name="s")

    @functools.partial(
        pl.kernel,
        mesh=mesh,
        out_type=jax.ShapeDtypeStruct((NC, n_pass * seg, D), _f32),
        scratch_types=[
            pltpu.VMEM_SHARED((seg, D), _f32),
            pltpu.VMEM((C, D), _f32),
            pltpu.VMEM((ZR, D), _f32),
            pltpu.VMEM((C,), _i32),
            pltpu.VMEM((C,), _i32),
            pltpu.VMEM((C,), _i32),
            pltpu.SemaphoreType.DMA,
        ],
    )
    def k(table_h, src_h, dst_h, out_h, shared, rows_v, zbuf, sidx, didx,
          lidx, sem):
        cid = lax.axis_index("c")
        sid = lax.axis_index("s")
        wid = sid * NC + cid
        base0 = wid * epw
        zeros16 = jnp.zeros((LANES,), _f32)

        @pl.loop(0, ZR)
        def _(r):
            for c8 in range(D // LANES):
                zbuf[r, pl.ds(c8 * LANES, LANES)] = zeros16

        for p in range(n_pass):
            nreal = min(n_out - p * half, half)
            for t in range(zchunk // ZR):
                pltpu.sync_copy(
                    zbuf, shared.at[pl.ds(sid * zchunk + t * ZR, ZR), :])
            plsc.subcore_barrier()

            @pl.loop(0, nch)
            def _(kc):
                base = base0 + kc * C
                pltpu.sync_copy(src_h.at[pl.ds(base, C)], sidx)
                pltpu.sync_copy(dst_h.at[pl.ds(base, C)], didx)
                for j in range(C // LANES):
                    d = didx[pl.ds(j * LANES, LANES)]
                    local = d - (p * half)
                    ok = (local >= 0) & (local < nreal)
                    lidx[pl.ds(j * LANES, LANES)] = jnp.where(ok, local, trash)
                pltpu.async_copy(table_h.at[sidx], rows_v, sem).wait()
                pltpu.sync_copy(rows_v, shared.at[lidx], add=True)

            plsc.subcore_barrier()
            for t in range(zchunk // ZR):
                rr = sid * zchunk + t * ZR
                pltpu.sync_copy(shared.at[pl.ds(rr, ZR), :],
                                rows_v.at[pl.ds(0, ZR), :])
                pltpu.sync_copy(rows_v.at[pl.ds(0, ZR), :],
                                out_h.at[cid, pl.ds(p * seg + rr, ZR), :])
            plsc.subcore_barrier()

    return k(table, src, dst)


def _segmin_call(vpad, src, dst, n0):
    epw = src.shape[0] // NW
    nch = epw // C
    flat = n0 * NF

    mesh = plsc.VectorSubcoreMesh(core_axis_name="c", subcore_axis_name="s")

    @functools.partial(
        pl.kernel,
        mesh=mesh,
        out_type=jax.ShapeDtypeStruct((NW, flat), _f32),
        scratch_types=[
            pltpu.VMEM((flat,), _f32),
            pltpu.VMEM((C, 2 * NF), _f32),
            pltpu.VMEM((C,), _i32),
            pltpu.VMEM((C,), _i32),
            pltpu.SemaphoreType.DMA,
        ],
    )
    def k(v_h, src_h, dst_h, out_h, death, vrows, sidx, didx, sem):
        cid = lax.axis_index("c")
        sid = lax.axis_index("s")
        wid = sid * NC + cid
        base0 = wid * epw
        iota = lax.iota(_i32, LANES)
        sel = iota // NF
        lane8 = iota & (NF - 1)
        init16 = jnp.full((LANES,), 2.0, _f32)

        @pl.loop(0, flat // LANES)
        def _(i):
            death[pl.ds(i * LANES, LANES)] = init16

        @pl.loop(0, nch)
        def _(kc):
            base = base0 + kc * C
            pltpu.sync_copy(src_h.at[pl.ds(base, C)], sidx)
            pltpu.sync_copy(dst_h.at[pl.ds(base, C)], didx)
            pltpu.async_copy(v_h.at[sidx], vrows, sem).wait()

            @pl.loop(0, C // 2)
            def _(g):
                e_a = 2 * g + sel
                e_b = 2 * g + (1 - sel)
                dpair = plsc.load_gather(didx, [e_a])
                dswap = plsc.load_gather(didx, [e_b])
                vvals = plsc.load_gather(vrows, [e_a, lane8])
                vswap = plsc.load_gather(vrows, [e_b, lane8])
                eq = dpair == dswap
                valid = dpair < n0
                vmin2 = jnp.where(eq, jnp.minimum(vvals, vswap), vvals)
                addr = jnp.where(valid, dpair, 0) * NF + lane8
                cur = plsc.load_gather(death, [addr])
                neww = jnp.minimum(cur, vmin2)
                mask = valid & ((sel == 0) | jnp.logical_not(eq))
                plsc.store_scatter(death, [addr], neww, mask=mask)

        pltpu.sync_copy(death, out_h.at[wid])

    return k(vpad, src, dst)


def _conv_call(x, au, ab, w1u, w2u, w1b, w2b, r_blk, au_map, ab_map,
               wf1=None, wf2p=None):
    n = x.shape[0]
    grid = n // r_blk
    with_v = wf1 is not None

    def body(*refs):
        i = 0
        x_r = refs[i]; i += 1
        au_r = ab_r = None
        if au is not None:
            au_r = refs[i]; i += 1
        if ab is not None:
            ab_r = refs[i]; i += 1
        w1u_r, w2u_r, w1b_r, w2b_r = refs[i:i + 4]; i += 4
        if with_v:
            wf1_r, wf2_r = refs[i:i + 2]; i += 2
        out_r = refs[i]; i += 1
        xb = x_r[...]
        xu = xb + (au_r[0] + au_r[1]) if au_r is not None else xb
        xbnd = xb + (ab_r[0] + ab_r[1]) if ab_r is not None else xb
        hu = jnp.dot(jax.nn.relu(jnp.dot(xu, w1u_r[...],
                     preferred_element_type=_f32)), w2u_r[...],
                     preferred_element_type=_f32)
        hb = jnp.dot(jax.nn.relu(jnp.dot(xbnd, w1b_r[...],
                     preferred_element_type=_f32)), w2b_r[...],
                     preferred_element_type=_f32)
        nb = jax.nn.relu(hu + hb)
        out_r[...] = nb
        if with_v:
            v_r = refs[i]
            t = jax.nn.relu(jnp.dot(nb, wf1_r[...], preferred_element_type=_f32))
            v_r[...] = jax.nn.sigmoid(jnp.dot(t, wf2_r[...],
                                              preferred_element_type=_f32))

    in_specs = [pl.BlockSpec((r_blk, D), lambda i: (i, 0))]
    args = [x]
    if au is not None:
        in_specs.append(pl.BlockSpec((NC, r_blk, D), au_map))
        args.append(au)
    if ab is not None:
        in_specs.append(pl.BlockSpec((NC, r_blk, D), ab_map))
        args.append(ab)
    wspec = pl.BlockSpec((D, D), lambda i: (0, 0))
    in_specs += [wspec] * 4
    args += [w1u, w2u, w1b, w2b]
    out_shape = [jax.ShapeDtypeStruct((n, D), _f32)]
    out_specs = [pl.BlockSpec((r_blk, D), lambda i: (i, 0))]
    if with_v:
        fh = wf1.shape[1]
        in_specs += [pl.BlockSpec((D, fh), lambda i: (0, 0)),
                     pl.BlockSpec((fh, 2 * NF), lambda i: (0, 0))]
        args += [wf1, wf2p]
        out_shape.append(jax.ShapeDtypeStruct((n, 2 * NF), _f32))
        out_specs.append(pl.BlockSpec((r_blk, 2 * NF), lambda i: (i, 0)))
    res = pl.pallas_call(
        body, grid=(grid,), in_specs=in_specs, out_specs=out_specs,
        out_shape=out_shape)(*args)
    return res if with_v else res[0]


def _rephine_call(mins, vpad, batch3, wd1, wd2, n0, r_blk):
    grid = n0 // r_blk
    fh = wd2.shape[0]

    def body(mins_r, v_r, b_r, wd1_r, wd2_r, out_r):
        i = pl.program_id(0)
        m = jnp.min(mins_r[...], axis=0)
        v8 = v_r[:, 0:NF]
        death = jnp.minimum(jnp.maximum(v8, m), 1.0)
        wd1v = wd1_r[0:1, :]
        wd1d = wd1_r[1:2, :]
        acc = jnp.zeros((r_blk, fh), _f32)
        for f in range(NF):
            t = jax.nn.relu(v8[:, f:f + 1] * wd1v + death[:, f:f + 1] * wd1d)
            acc = acc + jax.nn.relu(jnp.dot(t, wd2_r[...],
                                            preferred_element_type=_f32))
        b = b_r[0, 0, :]
        oh = (b[:, None] == lax.broadcasted_iota(_i32, (r_blk, BGRAPH), 1)
              ).astype(_f32)
        blockpool = jax.lax.dot_general(oh, acc, (((0,), (0,)), ((), ())),
                                        preferred_element_type=_f32)

        @pl.when(i == 0)
        def _():
            out_r[...] = blockpool

        @pl.when(i != 0)
        def _():
            out_r[...] = out_r[...] + blockpool

    return pl.pallas_call(
        body, grid=(grid,),
        in_specs=[pl.BlockSpec((NW, r_blk, NF), lambda i: (0, i, 0)),
                  pl.BlockSpec((r_blk, 2 * NF), lambda i: (i, 0)),
                  pl.BlockSpec((1, 1, r_blk), lambda i: (i, 0, 0)),
                  pl.BlockSpec((2, fh), lambda i: (0, 0)),
                  pl.BlockSpec((fh, fh), lambda i: (0, 0))],
        out_specs=pl.BlockSpec((BGRAPH, fh), lambda i: (0, 0)),
        out_shape=jax.ShapeDtypeStruct((BGRAPH, fh), _f32))(
            mins, vpad, batch3, wd1, wd2)


def _segpool_call(x, batch3, r_blk):
    n = x.shape[0]
    grid = n // r_blk

    def body(x_r, b_r, out_r):
        i = pl.program_id(0)
        b = b_r[0, 0, :]
        oh = (b[:, None] == lax.broadcasted_iota(_i32, (r_blk, BGRAPH), 1)
              ).astype(_f32)
        blockpool = jax.lax.dot_general(oh, x_r[...], (((0,), (0,)), ((), ())),
                                        preferred_element_type=_f32)

        @pl.when(i == 0)
        def _():
            out_r[...] = blockpool

        @pl.when(i != 0)
        def _():
            out_r[...] = out_r[...] + blockpool

    return pl.pallas_call(
        body, grid=(grid,),
        in_specs=[pl.BlockSpec((r_blk, D), lambda i: (i, 0)),
                  pl.BlockSpec((1, 1, r_blk), lambda i: (i, 0, 0))],
        out_specs=pl.BlockSpec((BGRAPH, D), lambda i: (0, 0)),
        out_shape=jax.ShapeDtypeStruct((BGRAPH, D), _f32))(x, batch3)


def _readout_call(p0, p1, p2, pools, Wph, Wlin1, blin1, Wlin2, blin2_2d):
    d2 = Wlin1.shape[2]
    oph = Wph.shape[2]
    ncls = Wlin2.shape[1]
    nl = Wph.shape[0]

    def body(p0_r, p1_r, p2_r, pools_r, wph_r, wlin1_r, blin1_r, wlin2_r,
             blin2_r, out_r):
        ps = (p0_r, p1_r, p2_r)
        x = jnp.zeros((BGRAPH, d2), _f32)
        for d in range(3):
            x = x + jax.nn.relu(jnp.dot(ps[d][...], wlin1_r[d],
                                        preferred_element_type=_f32)
                                + blin1_r[d:d + 1, :])
        ph = jnp.zeros((BGRAPH, oph), _f32)
        for l in range(nl):
            ph = ph + jnp.dot(pools_r[l], wph_r[l],
                              preferred_element_type=_f32)
        ph = ph * (1.0 / nl)
        out_r[...] = (jnp.dot(x, wlin2_r[0:d2, :], preferred_element_type=_f32)
                      + jnp.dot(ph, wlin2_r[d2:d2 + oph, :],
                                preferred_element_type=_f32)
                      + blin2_r[0:1, :])

    return pl.pallas_call(
        body, out_shape=jax.ShapeDtypeStruct((BGRAPH, ncls), _f32))(
            p0, p1, p2, pools, Wph, Wlin1, blin1, Wlin2, blin2_2d)


def kernel(x0, x1, x2, up_index0, up_index1, boundary_index1, boundary_index2,
           batch0, batch1, batch2, Wup1, Wup2, Wb1, Wb2, Wf1, Wf2, Wd1, Wd2,
           Wph, Wlin1, blin1, Wlin2, blin2):
    n0, n1, n2 = x0.shape[0], x1.shape[0], x2.shape[0]
    nl = Wup1.shape[0]
    fh = Wf1.shape[2]

    su0, du0 = _pad_edges(up_index0[0], up_index0[1])
    su1, du1 = _pad_edges(up_index1[0], up_index1[1])
    sb1, db1 = _pad_edges(boundary_index1[0], boundary_index1[1])
    sb2, db2 = _pad_edges(boundary_index2[0], boundary_index2[1])

    r0, r1, r2 = 400, 400, 200
    b0_3 = batch0.astype(_i32).reshape(n0 // r0, 1, r0)
    b1_3 = batch1.astype(_i32).reshape(n1 // r1, 1, r1)
    b2_3 = batch2.astype(_i32).reshape(n2 // r2, 1, r2)

    wf2p = jnp.concatenate(
        [Wf2, jnp.zeros((nl, fh, 2 * NF - Wf2.shape[2]), _f32)], axis=2)
    blin2_2d = blin2.reshape(1, -1)

    seg0, half0 = 10400, 10000
    seg1, half1 = 10400, 10000
    seg2, half2 = 5200, 5200

    au0_map = lambda i: (0, i, 0)
    au1_map = lambda i: (0, (i // 25) * (seg1 // r1) + i % 25, 0)
    ab2_map = lambda i: (0, i, 0)

    pools = []
    for l in range(nl):
        a_up0 = _segsum_call(x0, su0, du0, n0, half0, seg0)
        a_up1 = _segsum_call(x1, su1, du1, n1, half1, seg1)
        a_b1 = _segsum_call(x0, sb1, db1, n1, half1, seg1)
        a_b2 = _segsum_call(x1, sb2, db2, n2, half2, seg2)

        x0, vpad = _conv_call(x0, a_up0, None, Wup1[l, 0], Wup2[l, 0],
                              Wb1[l, 0], Wb2[l, 0], r0, au0_map, None,
                              wf1=Wf1[l], wf2p=wf2p[l])
        x1 = _conv_call(x1, a_up1, a_b1, Wup1[l, 1], Wup2[l, 1],
                        Wb1[l, 1], Wb2[l, 1], r1, au1_map, au1_map)
        x2 = _conv_call(x2, None, a_b2, Wup1[l, 2], Wup2[l, 2],
                        Wb1[l, 2], Wb2[l, 2], r2, None, ab2_map)

        mins = _segmin_call(vpad, su0, du0, n0).reshape(NW, n0, NF)
        pools.append(_rephine_call(mins, vpad, b0_3, Wd1[l], Wd2[l], n0, r0))

    p0 = _segpool_call(x0, b0_3, r0)
    p1 = _segpool_call(x1, b1_3, r1)
    p2 = _segpool_call(x2, b2_3, r2)
    return _readout_call(p0, p1, p2, jnp.stack(pools), Wph, Wlin1, blin1,
                         Wlin2, blin2_2d)


# speedup vs baseline: 2.5922x; 2.5922x over previous
import functools

import jax
import jax.numpy as jnp
from jax import lax
from jax.experimental import pallas as pl
from jax.experimental.pallas import tpu as pltpu
from jax.experimental.pallas import tpu_sc as plsc

NC, NS, LANES = 2, 16, 16
NW = NC * NS
C = 128
D = 128
NF = 8
BGRAPH = 64
ZR = 80

_f32 = jnp.float32
_i32 = jnp.int32


def _pad_edges(src, dst):
    e = src.shape[0]
    epad = -e % (NW * C)
    src = jnp.concatenate([src.astype(_i32), jnp.zeros((epad,), _i32)])
    dst = jnp.concatenate([dst.astype(_i32), jnp.full((epad,), 1 << 29, _i32)])
    return src, dst


def _segsum_call(table, src, dst, n_out, half, seg):
    n_pass = (n_out + half - 1) // half
    epw = src.shape[0] // NW
    nch = epw // C
    trash = seg - 1
    zchunk = seg // NS
    assert seg % NS == 0 and zchunk % ZR == 0

    mesh = plsc.VectorSubcoreMesh(core_axis_name="c", subcore_axis_name="s")

    @functools.partial(
        pl.kernel,
        mesh=mesh,
        out_type=jax.ShapeDtypeStruct((NC, n_pass * seg, D), _f32),
        scratch_types=[
            pltpu.VMEM_SHARED((seg, D), _f32),
            pltpu.VMEM((C, D), _f32),
            pltpu.VMEM((ZR, D), _f32),
            pltpu.VMEM((C,), _i32),
            pltpu.VMEM((C,), _i32),
            pltpu.VMEM((C,), _i32),
            pltpu.SemaphoreType.DMA,
        ],
    )
    def k(table_h, src_h, dst_h, out_h, shared, rows_v, zbuf, sidx, didx,
          lidx, sem):
        cid = lax.axis_index("c")
        sid = lax.axis_index("s")
        wid = sid * NC + cid
        base0 = wid * epw
        zeros16 = jnp.zeros((LANES,), _f32)

        @pl.loop(0, ZR)
        def _(r):
            for c8 in range(D // LANES):
                zbuf[r, pl.ds(c8 * LANES, LANES)] = zeros16

        for p in range(n_pass):
            nreal = min(n_out - p * half, half)
            for t in range(zchunk // ZR):
                pltpu.sync_copy(
                    zbuf, shared.at[pl.ds(sid * zchunk + t * ZR, ZR), :])
            plsc.subcore_barrier()

            @pl.loop(0, nch)
            def _(kc):
                base = base0 + kc * C
                pltpu.sync_copy(src_h.at[pl.ds(base, C)], sidx)
                pltpu.sync_copy(dst_h.at[pl.ds(base, C)], didx)
                for j in range(C // LANES):
                    d = didx[pl.ds(j * LANES, LANES)]
                    local = d - (p * half)
                    ok = (local >= 0) & (local < nreal)
                    lidx[pl.ds(j * LANES, LANES)] = jnp.where(ok, local, trash)
                pltpu.async_copy(table_h.at[sidx], rows_v, sem).wait()
                pltpu.sync_copy(rows_v, shared.at[lidx], add=True)

            plsc.subcore_barrier()
            for t in range(zchunk // ZR):
                rr = sid * zchunk + t * ZR
                pltpu.sync_copy(shared.at[pl.ds(rr, ZR), :],
                                rows_v.at[pl.ds(0, ZR), :])
                pltpu.sync_copy(rows_v.at[pl.ds(0, ZR), :],
                                out_h.at[cid, pl.ds(p * seg + rr, ZR), :])
            plsc.subcore_barrier()

    return k(table, src, dst)


def _segmin_call(vflat, src, dst, n0):
    epw = src.shape[0] // NW
    nch = epw // C
    half = n0 // 2
    hrows = -(-half * NF // D)
    hrows = -(-hrows // 8) * 8

    mesh = plsc.VectorSubcoreMesh(core_axis_name="c", subcore_axis_name="s")

    @functools.partial(
        pl.kernel,
        mesh=mesh,
        out_type=jax.ShapeDtypeStruct((NW, 2 * hrows, D), _f32),
        compiler_params=pltpu.CompilerParams(needs_layout_passes=False),
        scratch_types=[
            pltpu.VMEM(vflat.shape, _f32),
            pltpu.VMEM((hrows, D), _f32),
            pltpu.VMEM((C,), _i32),
            pltpu.VMEM((C,), _i32),
        ],
    )
    def k(v_h, src_h, dst_h, out_h, vtab, death, sidx, didx):
        cid = lax.axis_index("c")
        sid = lax.axis_index("s")
        wid = sid * NC + cid
        base0 = wid * epw
        pltpu.sync_copy(v_h, vtab)

        for p in range(2):
            @pl.loop(0, hrows)
            def _(r):
                for c8 in range(D // LANES):
                    death[r, pl.ds(c8 * LANES, LANES)] = jnp.full(
                        (LANES,), 2.0, _f32)

            @pl.loop(0, nch)
            def _(kc):
                base = base0 + kc * C
                pltpu.sync_copy(src_h.at[pl.ds(base, C)], sidx)
                pltpu.sync_copy(dst_h.at[pl.ds(base, C)], didx)

                @pl.loop(0, C // 2)
                def _(g):
                    iota = lax.iota(_i32, LANES)
                    sel = iota // NF
                    lane8 = iota & (NF - 1)
                    e_a = 2 * g + sel
                    e_b = 2 * g + (1 - sel)
                    dpair = plsc.load_gather(didx, [e_a])
                    dswap = plsc.load_gather(didx, [e_b])
                    spair = plsc.load_gather(sidx, [e_a])
                    sswap = plsc.load_gather(sidx, [e_b])
                    vf = spair * NF + lane8
                    vvals = plsc.load_gather(
                        vtab, [lax.shift_right_logical(vf, 7), vf & (D - 1)])
                    vg = sswap * NF + lane8
                    vsw = plsc.load_gather(
                        vtab, [lax.shift_right_logical(vg, 7), vg & (D - 1)])
                    local = dpair - p * half
                    valid = (local >= 0) & (local < half)
                    eq = dpair == dswap
                    vmin2 = jnp.where(eq, jnp.minimum(vvals, vsw), vvals)
                    af = jnp.where(valid, local, 0) * NF + lane8
                    arow = lax.shift_right_logical(af, 7)
                    acol = af & (D - 1)
                    cur = plsc.load_gather(death, [arow, acol])
                    neww = jnp.minimum(cur, vmin2)
                    mask = valid & ((sel == 0) | jnp.logical_not(eq))
                    plsc.store_scatter(death, [arow, acol], neww, mask=mask)

            pltpu.sync_copy(death, out_h.at[wid, pl.ds(p * hrows, hrows), :])

    return k(vflat, src, dst)


def _conv_call(x, au, ab, w1u, w2u, w1b, w2b, r_blk, au_map, ab_map,
               wf1=None, wf2p=None):
    n = x.shape[0]
    grid = n // r_blk
    with_v = wf1 is not None

    def body(*refs):
        i = 0
        x_r = refs[i]; i += 1
        au_r = ab_r = None
        if au is not None:
            au_r = refs[i]; i += 1
        if ab is not None:
            ab_r = refs[i]; i += 1
        w1u_r, w2u_r, w1b_r, w2b_r = refs[i:i + 4]; i += 4
        if with_v:
            wf1_r, wf2_r = refs[i:i + 2]; i += 2
        out_r = refs[i]; i += 1
        xb = x_r[...]
        xu = xb + (au_r[0] + au_r[1]) if au_r is not None else xb
        xbnd = xb + (ab_r[0] + ab_r[1]) if ab_r is not None else xb
        hu = jnp.dot(jax.nn.relu(jnp.dot(xu, w1u_r[...],
                     preferred_element_type=_f32)), w2u_r[...],
                     preferred_element_type=_f32)
        hb = jnp.dot(jax.nn.relu(jnp.dot(xbnd, w1b_r[...],
                     preferred_element_type=_f32)), w2b_r[...],
                     preferred_element_type=_f32)
        nb = jax.nn.relu(hu + hb)
        out_r[...] = nb
        if with_v:
            v_r = refs[i]
            t = jax.nn.relu(jnp.dot(nb, wf1_r[...], preferred_element_type=_f32))
            v_r[...] = jax.nn.sigmoid(jnp.dot(t, wf2_r[...],
                                              preferred_element_type=_f32))

    in_specs = [pl.BlockSpec((r_blk, D), lambda i: (i, 0))]
    args = [x]
    if au is not None:
        in_specs.append(pl.BlockSpec((NC, r_blk, D), au_map))
        args.append(au)
    if ab is not None:
        in_specs.append(pl.BlockSpec((NC, r_blk, D), ab_map))
        args.append(ab)
    wspec = pl.BlockSpec((D, D), lambda i: (0, 0))
    in_specs += [wspec] * 4
    args += [w1u, w2u, w1b, w2b]
    out_shape = [jax.ShapeDtypeStruct((n, D), _f32)]
    out_specs = [pl.BlockSpec((r_blk, D), lambda i: (i, 0))]
    if with_v:
        fh = wf1.shape[1]
        in_specs += [pl.BlockSpec((D, fh), lambda i: (0, 0)),
                     pl.BlockSpec((fh, 2 * NF), lambda i: (0, 0))]
        args += [wf1, wf2p]
        out_shape.append(jax.ShapeDtypeStruct((n, 2 * NF), _f32))
        out_specs.append(pl.BlockSpec((r_blk, 2 * NF), lambda i: (i, 0)))
    res = pl.pallas_call(
        body, grid=(grid,), in_specs=in_specs, out_specs=out_specs,
        out_shape=out_shape)(*args)
    return res if with_v else res[0]


def _rephine_call(mins, vpad, batch3, wd1, wd2, n0, r_blk):
    grid = n0 // r_blk
    fh = wd2.shape[0]

    def body(mins_r, v_r, b_r, wd1_r, wd2_r, out_r):
        i = pl.program_id(0)
        m = jnp.min(mins_r[...], axis=0)
        v8 = v_r[:, 0:NF]
        death = jnp.minimum(jnp.maximum(v8, m), 1.0)
        wd1v = wd1_r[0:1, :]
        wd1d = wd1_r[1:2, :]
        acc = jnp.zeros((r_blk, fh), _f32)
        for f in range(NF):
            t = jax.nn.relu(v8[:, f:f + 1] * wd1v + death[:, f:f + 1] * wd1d)
            acc = acc + jax.nn.relu(jnp.dot(t, wd2_r[...],
                                            preferred_element_type=_f32))
        b = b_r[0, 0, :]
        oh = (b[:, None] == lax.broadcasted_iota(_i32, (r_blk, BGRAPH), 1)
              ).astype(_f32)
        blockpool = jax.lax.dot_general(oh, acc, (((0,), (0,)), ((), ())),
                                        preferred_element_type=_f32)

        @pl.when(i == 0)
        def _():
            out_r[...] = blockpool

        @pl.when(i != 0)
        def _():
            out_r[...] = out_r[...] + blockpool

    return pl.pallas_call(
        body, grid=(grid,),
        in_specs=[pl.BlockSpec((NW, r_blk, NF), lambda i: (0, i, 0)),
                  pl.BlockSpec((r_blk, 2 * NF), lambda i: (i, 0)),
                  pl.BlockSpec((1, 1, r_blk), lambda i: (i, 0, 0)),
                  pl.BlockSpec((2, fh), lambda i: (0, 0)),
                  pl.BlockSpec((fh, fh), lambda i: (0, 0))],
        out_specs=pl.BlockSpec((BGRAPH, fh), lambda i: (0, 0)),
        out_shape=jax.ShapeDtypeStruct((BGRAPH, fh), _f32))(
            mins, vpad, batch3, wd1, wd2)


def _segpool_call(x, batch3, r_blk):
    n = x.shape[0]
    grid = n // r_blk

    def body(x_r, b_r, out_r):
        i = pl.program_id(0)
        b = b_r[0, 0, :]
        oh = (b[:, None] == lax.broadcasted_iota(_i32, (r_blk, BGRAPH), 1)
              ).astype(_f32)
        blockpool = jax.lax.dot_general(oh, x_r[...], (((0,), (0,)), ((), ())),
                                        preferred_element_type=_f32)

        @pl.when(i == 0)
        def _():
            out_r[...] = blockpool

        @pl.when(i != 0)
        def _():
            out_r[...] = out_r[...] + blockpool

    return pl.pallas_call(
        body, grid=(grid,),
        in_specs=[pl.BlockSpec((r_blk, D), lambda i: (i, 0)),
                  pl.BlockSpec((1, 1, r_blk), lambda i: (i, 0, 0))],
        out_specs=pl.BlockSpec((BGRAPH, D), lambda i: (0, 0)),
        out_shape=jax.ShapeDtypeStruct((BGRAPH, D), _f32))(x, batch3)


def _readout_call(p0, p1, p2, pools, Wph, Wlin1, blin1, Wlin2, blin2_2d):
    d2 = Wlin1.shape[2]
    oph = Wph.shape[2]
    ncls = Wlin2.shape[1]
    nl = Wph.shape[0]

    def body(p0_r, p1_r, p2_r, pools_r, wph_r, wlin1_r, blin1_r, wlin2_r,
             blin2_r, out_r):
        ps = (p0_r, p1_r, p2_r)
        x = jnp.zeros((BGRAPH, d2), _f32)
        for d in range(3):
            x = x + jax.nn.relu(jnp.dot(ps[d][...], wlin1_r[d],
                                        preferred_element_type=_f32)
                                + blin1_r[d:d + 1, :])
        ph = jnp.zeros((BGRAPH, oph), _f32)
        for l in range(nl):
            ph = ph + jnp.dot(pools_r[l], wph_r[l],
                              preferred_element_type=_f32)
        ph = ph * (1.0 / nl)
        out_r[...] = (jnp.dot(x, wlin2_r[0:d2, :], preferred_element_type=_f32)
                      + jnp.dot(ph, wlin2_r[d2:d2 + oph, :],
                                preferred_element_type=_f32)
                      + blin2_r[0:1, :])

    return pl.pallas_call(
        body, out_shape=jax.ShapeDtypeStruct((BGRAPH, ncls), _f32))(
            p0, p1, p2, pools, Wph, Wlin1, blin1, Wlin2, blin2_2d)


def kernel(x0, x1, x2, up_index0, up_index1, boundary_index1, boundary_index2,
           batch0, batch1, batch2, Wup1, Wup2, Wb1, Wb2, Wf1, Wf2, Wd1, Wd2,
           Wph, Wlin1, blin1, Wlin2, blin2):
    n0, n1, n2 = x0.shape[0], x1.shape[0], x2.shape[0]
    nl = Wup1.shape[0]
    fh = Wf1.shape[2]

    su0, du0 = _pad_edges(up_index0[0], up_index0[1])
    su1, du1 = _pad_edges(up_index1[0], up_index1[1])
    sb1, db1 = _pad_edges(boundary_index1[0], boundary_index1[1])
    sb2, db2 = _pad_edges(boundary_index2[0], boundary_index2[1])

    r0, r1, r2 = 400, 400, 200
    b0_3 = batch0.astype(_i32).reshape(n0 // r0, 1, r0)
    b1_3 = batch1.astype(_i32).reshape(n1 // r1, 1, r1)
    b2_3 = batch2.astype(_i32).reshape(n2 // r2, 1, r2)

    wf2p = jnp.concatenate(
        [Wf2, jnp.zeros((nl, fh, 2 * NF - Wf2.shape[2]), _f32)], axis=2)
    blin2_2d = blin2.reshape(1, -1)

    seg0, half0 = 12800, 10000
    seg1, half1 = 12800, 10000
    seg2, half2 = 6400, 6400

    au0_map = lambda i: (0, i, 0)
    au1_map = lambda i: (0, (i // 25) * (seg1 // r1) + i % 25, 0)
    ab2_map = lambda i: (0, i, 0)

    pools = []
    for l in range(nl):
        a_up0 = _segsum_call(x0, su0, du0, n0, half0, seg0)
        a_up1 = _segsum_call(x1, su1, du1, n1, half1, seg1)
        a_b1 = _segsum_call(x0, sb1, db1, n1, half1, seg1)
        a_b2 = _segsum_call(x1, sb2, db2, n2, half2, seg2)

        x0, vpad = _conv_call(x0, a_up0, None, Wup1[l, 0], Wup2[l, 0],
                              Wb1[l, 0], Wb2[l, 0], r0, au0_map, None,
                              wf1=Wf1[l], wf2p=wf2p[l])
        x1 = _conv_call(x1, a_up1, a_b1, Wup1[l, 1], Wup2[l, 1],
                        Wb1[l, 1], Wb2[l, 1], r1, au1_map, au1_map)
        x2 = _conv_call(x2, None, a_b2, Wup1[l, 2], Wup2[l, 2],
                        Wb1[l, 2], Wb2[l, 2], r2, None, ab2_map)

        vflat = vpad[:, :NF].reshape(n0 * NF // D, D)
        mraw = _segmin_call(vflat, su0, du0, n0)
        hrows = mraw.shape[1] // 2
        mins = mraw.reshape(NW, 2, hrows * D)[:, :, :n0 // 2 * NF]
        mins = mins.reshape(NW, n0, NF)
        pools.append(_rephine_call(mins, vpad, b0_3, Wd1[l], Wd2[l], n0, r0))

    p0 = _segpool_call(x0, b0_3, r0)
    p1 = _segpool_call(x1, b1_3, r1)
    p2 = _segpool_call(x2, b2_3, r2)
    return _readout_call(p0, p1, p2, jnp.stack(pools), Wph, Wlin1, blin1,
                         Wlin2, blin2_2d)
